# Initial kernel scaffold; baseline (speedup 1.0000x reference)
import jax
import jax.numpy as jnp
from jax.experimental import pallas as pl


def kernel(spatial_pos, table):
    raise NotImplementedError("write your pallas kernel here")


# speedup vs baseline: 7.7926x; 7.7926x over previous
import jax
import jax.numpy as jnp
from jax import lax
from jax.experimental import pallas as pl
from jax.experimental.pallas import tpu as pltpu
from jax.experimental.pallas import tpu_sc as plsc

B = 16
H = 16
N = 512
NP1 = N + 1
V = 245
VP = 256
HPT = 8
RB = 8
NBLK = N // RB


def _sc_body(sp_hbm, tbl_hbm, out_hbm, tbl_v, sp_v, out_v, zero_v):
    b = lax.axis_index("s")
    c = lax.axis_index("c")
    h0 = c * HPT

    pltpu.sync_copy(tbl_hbm.at[pl.ds(h0 * VP, HPT * VP)], tbl_v)

    zeros = jnp.zeros((16,), jnp.float32)
    for h in range(HPT):
        for r in range(RB):
            out_v[h, r, pl.ds(0, 16)] = zeros
    for j in range(N // 16):
        zero_v[pl.ds(j * 16, 16)] = zeros
    zero_v[pl.ds(NP1 - 16, 16)] = zeros
    for h in range(HPT):
        pltpu.sync_copy(zero_v, out_hbm.at[b, h0 + h, 0, :])

    hoff = [jnp.full((16,), h * VP, jnp.int32) for h in range(HPT)]

    def blk_body(blk):
        pltpu.sync_copy(sp_hbm.at[b, pl.ds(blk * RB, RB), :], sp_v)

        def row_body(r):
            for j in range(N // 16):
                idx = sp_v[r, pl.ds(j * 16, 16)]
                for h in range(HPT):
                    vals = plsc.load_gather(tbl_v, [hoff[h] + idx])
                    out_v[h, r, pl.ds(1 + j * 16, 16)] = vals

        pl.loop(0, RB)(row_body)

        for h in range(HPT):
            pltpu.sync_copy(out_v.at[h],
                            out_hbm.at[b, h0 + h, pl.ds(blk * RB + 1, RB), :])

    pl.loop(0, NBLK)(blk_body)


@jax.jit
def kernel(spatial_pos, table):
    sp = spatial_pos.astype(jnp.int32)
    tblT = jnp.zeros((H, VP), jnp.float32).at[:, :V].set(table.T).reshape(H * VP)

    mesh = plsc.VectorSubcoreMesh(core_axis_name="c", subcore_axis_name="s")
    run = pl.kernel(
        _sc_body,
        out_type=jax.ShapeDtypeStruct((B, H, NP1, NP1), jnp.float32),
        mesh=mesh,
        compiler_params=pltpu.CompilerParams(
            use_tc_tiling_on_sc=False, needs_layout_passes=False),
        scratch_types=[
            pltpu.VMEM((HPT * VP,), jnp.float32),
            pltpu.VMEM((RB, N), jnp.int32),
            pltpu.VMEM((HPT, RB, NP1), jnp.float32),
            pltpu.VMEM((NP1,), jnp.float32),
        ],
    )
    return run(sp, tblT)
